# Initial kernel scaffold; baseline (speedup 1.0000x reference)
import jax
import jax.numpy as jnp
from jax.experimental import pallas as pl


def kernel(x, gate_W, gate_b, expert_W, expert_b):
    raise NotImplementedError("write your pallas kernel here")


# speedup vs baseline: 2.1859x; 2.1859x over previous
import jax
import jax.numpy as jnp
from jax.experimental import pallas as pl
from jax.experimental.pallas import tpu as pltpu


def _moe_block(x_ref, gw_ref, gb_ref, ew_ref, eb_ref, out_ref, *, num_experts):
    e = pl.program_id(1)
    x = x_ref[...]
    logits = jnp.dot(x, gw_ref[...], preferred_element_type=jnp.float32)
    logits = logits + gb_ref[...]
    g = jax.nn.softmax(logits, axis=-1)
    onehot = (jax.lax.broadcasted_iota(jnp.int32, (1, num_experts), 1) == e)
    w = jnp.sum(g * onehot.astype(jnp.float32), axis=1, keepdims=True)
    y = jnp.dot(x, ew_ref[0], preferred_element_type=jnp.float32)
    y = y + eb_ref[0]
    contrib = w * y

    @pl.when(e == 0)
    def _init():
        out_ref[...] = contrib

    @pl.when(e != 0)
    def _acc():
        out_ref[...] += contrib


def kernel(x, gate_W, gate_b, expert_W, expert_b):
    tokens, d = x.shape
    num_experts, _, f = expert_W.shape
    bt = min(2048, tokens)
    grid = (tokens // bt, num_experts)

    gate_b2 = gate_b.reshape(1, num_experts)
    expert_b3 = expert_b.reshape(num_experts, 1, f)

    return pl.pallas_call(
        lambda *refs: _moe_block(*refs, num_experts=num_experts),
        grid=grid,
        in_specs=[
            pl.BlockSpec((bt, d), lambda i, e: (i, 0)),
            pl.BlockSpec((d, num_experts), lambda i, e: (0, 0)),
            pl.BlockSpec((1, num_experts), lambda i, e: (0, 0)),
            pl.BlockSpec((1, d, f), lambda i, e: (e, 0, 0)),
            pl.BlockSpec((1, 1, f), lambda i, e: (e, 0, 0)),
        ],
        out_specs=pl.BlockSpec((bt, f), lambda i, e: (i, 0)),
        out_shape=jax.ShapeDtypeStruct((tokens, f), jnp.float32),
        compiler_params=pltpu.CompilerParams(
            dimension_semantics=("parallel", "arbitrary"),
        ),
    )(x, gate_W, gate_b2, expert_W, expert_b3)
